# baseline (speedup 1.0000x reference)
import functools

import jax
import jax.numpy as jnp
from jax import lax
from jax.experimental import pallas as pl
from jax.experimental.pallas import tpu as pltpu
from jax.experimental.pallas import tpu_sc as plsc

N = 10000
E = 320000
D = 128
G = 16

NC = 2
NS = 16
NW = NC * NS
E2 = E // 2
EW = E2 // NW
CH = 64
NCHUNK = EW // CH
TAILE = EW - NCHUNK * CH
STRIPE = 624
TAIL = N - NS * STRIPE


def _prep_body(x_ref, ga_ref, wmsg_ref, wnode_ref, bmsg_ref, bnode_ref,
               xs_ref, xd_ref, xn_ref, gb_ref):
    x = x_ref[...]
    xs_ref[...] = jnp.dot(x, wmsg_ref[0:D, :], preferred_element_type=jnp.float32)
    xd_ref[...] = jnp.dot(x, wmsg_ref[D:2 * D, :],
                          preferred_element_type=jnp.float32) + bmsg_ref[...]
    xn_ref[...] = jnp.dot(x, wnode_ref[0:D, :],
                          preferred_element_type=jnp.float32) + bnode_ref[...]
    gb_ref[...] = jnp.dot(ga_ref[...], wnode_ref[2 * D:3 * D, :],
                          preferred_element_type=jnp.float32)


def _prep(x, graph_attr, W_msg, W_node, b_msg, b_node):
    return pl.pallas_call(
        _prep_body,
        out_shape=[
            jax.ShapeDtypeStruct((N, D), jnp.float32),
            jax.ShapeDtypeStruct((N, D), jnp.float32),
            jax.ShapeDtypeStruct((N, D), jnp.float32),
            jax.ShapeDtypeStruct((G, D), jnp.float32),
        ],
    )(x, graph_attr, W_msg, W_node, b_msg.reshape(1, D), b_node.reshape(1, D))


_BE = 4000


def _edge_mm_body(ea_ref, w_ref, a_ref):
    a_ref[...] = jnp.dot(ea_ref[...], w_ref[...],
                         preferred_element_type=jnp.float32)


def _edge_mm(edge_attr_half, W3):
    grid = E2 // _BE
    return pl.pallas_call(
        _edge_mm_body,
        grid=(grid,),
        in_specs=[
            pl.BlockSpec((_BE, D), lambda i: (i, 0)),
            pl.BlockSpec((D, D), lambda i: (0, 0)),
        ],
        out_specs=pl.BlockSpec((_BE, D), lambda i: (i, 0)),
        out_shape=jax.ShapeDtypeStruct((E2, D), jnp.float32),
        compiler_params=pltpu.CompilerParams(
            dimension_semantics=("arbitrary",)),
    )(edge_attr_half, W3)


def _sc_edge_body(off, a_hbm, xs_hbm, xd_hbm, src_hbm, dst_hbm,
                  ee_hbm, aggp_hbm,
                  src_v0, dst_v0, a_v0, g1_v0, g2_v0,
                  src_v1, dst_v1, a_v1, g1_v1, g2_v1,
                  tsrc_v, tdst_v, agg_sh,
                  sa0, sg10, sg20, sa1, sg11, sg21,
                  we0, ws0, we1, ws1, si0, si1):
    c = lax.axis_index("c")
    s = lax.axis_index("s")
    wid = c * NS + s
    edge_base = wid * EW

    src_v = (src_v0, src_v1)
    dst_v = (dst_v0, dst_v1)
    a_v = (a_v0, a_v1)
    g1_v = (g1_v0, g1_v1)
    g2_v = (g2_v0, g2_v1)
    sa = (sa0, sa1)
    sg1 = (sg10, sg11)
    sg2 = (sg20, sg21)
    we = (we0, we1)
    ws = (ws0, ws1)
    si = (si0, si1)

    zv = jnp.zeros((16,), jnp.float32)

    def _zero_row(r, carry):
        for cc in range(D // 16):
            g1_v0[r, pl.ds(cc * 16, 16)] = zv
        return carry

    lax.fori_loop(0, CH, _zero_row, 0)
    for j in range(STRIPE // CH):
        pltpu.sync_copy(g1_v0, agg_sh.at[pl.ds(s * STRIPE + j * CH, CH)])
    rem = STRIPE - (STRIPE // CH) * CH
    if rem:
        pltpu.sync_copy(g1_v0.at[pl.ds(0, rem)],
                        agg_sh.at[pl.ds(s * STRIPE + STRIPE - rem, rem)])

    @pl.when(s == NS - 1)
    def _zero_tail():
        pltpu.sync_copy(g1_v0.at[pl.ds(0, TAIL)],
                        agg_sh.at[pl.ds(NS * STRIPE, TAIL)])

    plsc.subcore_barrier()

    def _idx_start(i, b):
        base = edge_base + i * CH
        pltpu.async_copy(src_hbm.at[pl.ds(base, CH)], src_v[b], si[b])
        pltpu.async_copy(dst_hbm.at[pl.ds(base, CH)], dst_v[b], si[b])

    def _idx_wait(b):
        pltpu.make_async_copy(src_hbm.at[pl.ds(0, CH)], src_v[b], si[b]).wait()
        pltpu.make_async_copy(dst_hbm.at[pl.ds(0, CH)], dst_v[b], si[b]).wait()

    def _gather_start(i, b):
        base = edge_base + i * CH
        pltpu.async_copy(a_hbm.at[pl.ds(base, CH)], a_v[b], sa[b])
        pltpu.async_copy(xs_hbm.at[src_v[b]], g1_v[b], sg1[b])
        pltpu.async_copy(xd_hbm.at[dst_v[b]], g2_v[b], sg2[b])

    def _gather_wait(b):
        pltpu.make_async_copy(a_hbm.at[pl.ds(0, CH)], a_v[b], sa[b]).wait()
        pltpu.make_async_copy(xs_hbm.at[src_v[b]], g1_v[b], sg1[b]).wait()
        pltpu.make_async_copy(xd_hbm.at[dst_v[b]], g2_v[b], sg2[b]).wait()

    def _wb_start(i, b):
        base = edge_base + i * CH
        pltpu.async_copy(a_v[b], ee_hbm.at[pl.ds(off + base, CH)], we[b])
        pltpu.async_copy(a_v[b], agg_sh.at[dst_v[b]], ws[b], add=True)

    def _wb_wait(b):
        pltpu.make_async_copy(a_v[b], ee_hbm.at[pl.ds(0, CH)], we[b]).wait()
        pltpu.make_async_copy(a_v[b], agg_sh.at[dst_v[b]], ws[b]).wait()

    def _compute(b, nrows):
        ab, g1b, g2b = a_v[b], g1_v[b], g2_v[b]

        def _row2(r2, carry2):
            r = r2 * 2
            for dr in range(2):
                for cc in range(D // 16):
                    sl = pl.ds(cc * 16, 16)
                    ab[r + dr, sl] = jnp.maximum(
                        ab[r + dr, sl] + g1b[r + dr, sl] + g2b[r + dr, sl],
                        0.0)
            return carry2

        lax.fori_loop(0, nrows // 2, _row2, 0)
        if nrows % 2:
            r = nrows - 1
            for cc in range(D // 16):
                sl = pl.ds(cc * 16, 16)
                ab[r, sl] = jnp.maximum(
                    ab[r, sl] + g1b[r, sl] + g2b[r, sl], 0.0)

    _idx_start(0, 0)
    _idx_wait(0)
    _gather_start(0, 0)

    @pl.loop(0, NCHUNK, step=2)
    def _pair(i0):
        for b in (0, 1):
            i = i0 + b
            nxt = 1 - b

            @pl.when(i > 0)
            def _():
                _wb_wait(nxt)

            @pl.when(i + 1 < NCHUNK)
            def _():
                _idx_start(i + 1, nxt)

            _gather_wait(b)

            @pl.when(i + 1 < NCHUNK)
            def _():
                _idx_wait(nxt)
                _gather_start(i + 1, nxt)

            _compute(b, CH)
            _wb_start(i, b)

    _wb_wait((NCHUNK - 1) % 2)

    if TAILE:
        base = edge_base + NCHUNK * CH
        pltpu.sync_copy(src_hbm.at[pl.ds(base, TAILE)], tsrc_v)
        pltpu.sync_copy(dst_hbm.at[pl.ds(base, TAILE)], tdst_v)
        pltpu.sync_copy(a_hbm.at[pl.ds(base, TAILE)],
                        a_v0.at[pl.ds(0, TAILE)])
        pltpu.async_copy(xs_hbm.at[tsrc_v], g1_v0.at[pl.ds(0, TAILE)],
                         sg10).wait()
        pltpu.async_copy(xd_hbm.at[tdst_v], g2_v0.at[pl.ds(0, TAILE)],
                         sg20).wait()
        _compute(0, TAILE)
        pltpu.sync_copy(a_v0.at[pl.ds(0, TAILE)],
                        ee_hbm.at[pl.ds(off + base, TAILE)])
        pltpu.sync_copy(a_v0.at[pl.ds(0, TAILE)], agg_sh.at[tdst_v],
                        add=True)

    plsc.subcore_barrier()
    pltpu.sync_copy(agg_sh.at[pl.ds(s * STRIPE, STRIPE)],
                    aggp_hbm.at[c, pl.ds(s * STRIPE, STRIPE)])

    @pl.when(s == NS - 1)
    def _dump_tail():
        pltpu.sync_copy(agg_sh.at[pl.ds(NS * STRIPE, TAIL)],
                        aggp_hbm.at[c, pl.ds(NS * STRIPE, TAIL)])


_SC_SCRATCH = (
    [pltpu.VMEM((CH,), jnp.int32),
     pltpu.VMEM((CH,), jnp.int32),
     pltpu.VMEM((CH, D), jnp.float32),
     pltpu.VMEM((CH, D), jnp.float32),
     pltpu.VMEM((CH, D), jnp.float32)] * 2
    + [pltpu.VMEM((TAILE, ), jnp.int32),
       pltpu.VMEM((TAILE, ), jnp.int32),
       pltpu.VMEM_SHARED((N, D), jnp.float32)]
    + [pltpu.SemaphoreType.DMA] * 12
)


def _sc_edge_first(a, xs, xd, src, dst):
    k = pl.kernel(
        functools.partial(_sc_edge_body, 0),
        mesh=plsc.VectorSubcoreMesh(core_axis_name="c", subcore_axis_name="s"),
        out_type=[
            jax.ShapeDtypeStruct((E, D), jnp.float32),
            jax.ShapeDtypeStruct((NC, N, D), jnp.float32),
        ],
        scratch_types=_SC_SCRATCH,
    )
    return k(a, xs, xd, src, dst)


def _sc_edge_second(a, xs, xd, src, dst, ee):
    def body(a_hbm, xs_hbm, xd_hbm, src_hbm, dst_hbm, ee_hbm, aggp_hbm,
             *rest):
        return _sc_edge_body(E2, a_hbm, xs_hbm, xd_hbm, src_hbm, dst_hbm,
                             ee_hbm, aggp_hbm, *rest)

    k = pl.kernel(
        body,
        mesh=plsc.VectorSubcoreMesh(core_axis_name="c", subcore_axis_name="s"),
        out_type=[
            jax.ShapeDtypeStruct((NC, N, D), jnp.float32),
        ],
        scratch_types=_SC_SCRATCH,
    )
    out = k(a, xs, xd, src, dst, ee)
    return out[0] if isinstance(out, (list, tuple)) else out


_BN = 1000


def _node_body(xn_ref, a0_ref, a1_ref, a2_ref, a3_ref, wn2_ref, gb_ref,
               b_ref, ne_ref, ge_ref):
    i = pl.program_id(0)
    agg = (a0_ref[...] + a1_ref[...]) + (a2_ref[...] + a3_ref[...])
    bvec = b_ref[0, 0, :]
    seg = jax.lax.broadcasted_iota(jnp.int32, (_BN, G), 1)
    mask = seg == bvec[:, None]
    gbb = jnp.dot(mask.astype(jnp.float32), gb_ref[...],
                  preferred_element_type=jnp.float32)
    ne = xn_ref[...] + jnp.dot(agg, wn2_ref[...],
                               preferred_element_type=jnp.float32) + gbb
    ne = jnp.maximum(ne, 0.0)
    ne_ref[...] = ne

    @pl.when(i == 0)
    def _():
        ge_ref[...] = jnp.full((G, D), -jnp.inf, jnp.float32)

    rows = [
        jnp.max(jnp.where(mask[:, g:g + 1], ne, -jnp.inf), axis=0,
                keepdims=True)
        for g in range(G)
    ]
    ge_ref[...] = jnp.maximum(ge_ref[...], jnp.concatenate(rows, axis=0))


def _node(xn, aggs, Wn2, gb, batch3):
    grid = N // _BN
    return pl.pallas_call(
        _node_body,
        grid=(grid,),
        in_specs=[
            pl.BlockSpec((_BN, D), lambda i: (i, 0)),
            pl.BlockSpec((_BN, D), lambda i: (i, 0)),
            pl.BlockSpec((_BN, D), lambda i: (i, 0)),
            pl.BlockSpec((_BN, D), lambda i: (i, 0)),
            pl.BlockSpec((_BN, D), lambda i: (i, 0)),
            pl.BlockSpec((D, D), lambda i: (0, 0)),
            pl.BlockSpec((G, D), lambda i: (0, 0)),
            pl.BlockSpec((1, 1, _BN), lambda i: (i, 0, 0)),
        ],
        out_specs=[
            pl.BlockSpec((_BN, D), lambda i: (i, 0)),
            pl.BlockSpec((G, D), lambda i: (0, 0)),
        ],
        out_shape=[
            jax.ShapeDtypeStruct((N, D), jnp.float32),
            jax.ShapeDtypeStruct((G, D), jnp.float32),
        ],
        compiler_params=pltpu.CompilerParams(
            dimension_semantics=("arbitrary",)),
    )(xn, aggs[0], aggs[1], aggs[2], aggs[3], Wn2, gb, batch3)


def kernel(x, edge_index, edge_attr, graph_attr, batch,
           W_msg, b_msg, W_node, b_node):
    src = edge_index[0]
    dst = edge_index[1]
    xs, xd, xn, gb = _prep(x, graph_attr, W_msg, W_node, b_msg, b_node)
    W3 = W_msg[2 * D:3 * D, :]
    a0 = _edge_mm(edge_attr[:E2], W3)
    a1 = _edge_mm(edge_attr[E2:], W3)
    ee, aggp0 = _sc_edge_first(a0, xs, xd, src[:E2], dst[:E2])
    aggp1 = _sc_edge_second(a1, xs, xd, src[E2:], dst[E2:], ee)
    ne, ge = _node(xn, (aggp0[0], aggp0[1], aggp1[0], aggp1[1]),
                   W_node[D:2 * D, :], gb,
                   batch.reshape(N // _BN, 1, _BN))
    return ne, ee, ge


# speedup vs baseline: 1.0001x; 1.0001x over previous
import functools

import jax
import jax.numpy as jnp
from jax import lax
from jax.experimental import pallas as pl
from jax.experimental.pallas import tpu as pltpu
from jax.experimental.pallas import tpu_sc as plsc

N = 10000
E = 320000
D = 128
G = 16

NC = 2
NS = 16
NW = NC * NS
E2 = E // 2
EW = E2 // NW
CH = 64
NCHUNK = EW // CH
TAILE = EW - NCHUNK * CH
STRIPE = 624
TAIL = N - NS * STRIPE


def _prep_body(x_ref, ga_ref, wmsg_ref, wnode_ref, bmsg_ref, bnode_ref,
               xs_ref, xd_ref, xn_ref, gb_ref):
    x = x_ref[...]
    xs_ref[...] = jnp.dot(x, wmsg_ref[0:D, :], preferred_element_type=jnp.float32)
    xd_ref[...] = jnp.dot(x, wmsg_ref[D:2 * D, :],
                          preferred_element_type=jnp.float32) + bmsg_ref[...]
    xn_ref[...] = jnp.dot(x, wnode_ref[0:D, :],
                          preferred_element_type=jnp.float32) + bnode_ref[...]
    gb_ref[...] = jnp.dot(ga_ref[...], wnode_ref[2 * D:3 * D, :],
                          preferred_element_type=jnp.float32)


def _prep(x, graph_attr, W_msg, W_node, b_msg, b_node):
    return pl.pallas_call(
        _prep_body,
        out_shape=[
            jax.ShapeDtypeStruct((N, D), jnp.float32),
            jax.ShapeDtypeStruct((N, D), jnp.float32),
            jax.ShapeDtypeStruct((N, D), jnp.float32),
            jax.ShapeDtypeStruct((G, D), jnp.float32),
        ],
    )(x, graph_attr, W_msg, W_node, b_msg.reshape(1, D), b_node.reshape(1, D))


_BE = 4000


def _edge_mm_body(ea_ref, w_ref, a_ref):
    a_ref[...] = jnp.dot(ea_ref[...], w_ref[...],
                         preferred_element_type=jnp.float32)


def _edge_mm(edge_attr_half, W3):
    grid = E2 // _BE
    return pl.pallas_call(
        _edge_mm_body,
        grid=(grid,),
        in_specs=[
            pl.BlockSpec((_BE, D), lambda i: (i, 0)),
            pl.BlockSpec((D, D), lambda i: (0, 0)),
        ],
        out_specs=pl.BlockSpec((_BE, D), lambda i: (i, 0)),
        out_shape=jax.ShapeDtypeStruct((E2, D), jnp.float32),
        compiler_params=pltpu.CompilerParams(
            dimension_semantics=("arbitrary",)),
    )(edge_attr_half, W3)


def _sc_edge_body(off, a_hbm, xs_hbm, xd_hbm, src_hbm, dst_hbm,
                  ee_hbm, aggp_hbm,
                  src_v0, dst_v0, a_v0, g1_v0, g2_v0,
                  src_v1, dst_v1, a_v1, g1_v1, g2_v1,
                  tsrc_v, tdst_v, agg_sh,
                  sa0, sg10, sg20, sa1, sg11, sg21,
                  we0, ws0, we1, ws1, si0, si1):
    c = lax.axis_index("c")
    s = lax.axis_index("s")
    wid = c * NS + s
    edge_base = wid * EW

    src_v = (src_v0, src_v1)
    dst_v = (dst_v0, dst_v1)
    a_v = (a_v0, a_v1)
    g1_v = (g1_v0, g1_v1)
    g2_v = (g2_v0, g2_v1)
    sa = (sa0, sa1)
    sg1 = (sg10, sg11)
    sg2 = (sg20, sg21)
    we = (we0, we1)
    ws = (ws0, ws1)
    si = (si0, si1)

    zv = jnp.zeros((16,), jnp.float32)

    def _zero_row(r, carry):
        for cc in range(D // 16):
            g1_v0[r, pl.ds(cc * 16, 16)] = zv
        return carry

    lax.fori_loop(0, CH, _zero_row, 0)
    for j in range(STRIPE // CH):
        pltpu.sync_copy(g1_v0, agg_sh.at[pl.ds(s * STRIPE + j * CH, CH)])
    rem = STRIPE - (STRIPE // CH) * CH
    if rem:
        pltpu.sync_copy(g1_v0.at[pl.ds(0, rem)],
                        agg_sh.at[pl.ds(s * STRIPE + STRIPE - rem, rem)])

    @pl.when(s == NS - 1)
    def _zero_tail():
        pltpu.sync_copy(g1_v0.at[pl.ds(0, TAIL)],
                        agg_sh.at[pl.ds(NS * STRIPE, TAIL)])

    plsc.subcore_barrier()

    def _idx_start(i, b):
        base = edge_base + i * CH
        pltpu.async_copy(src_hbm.at[pl.ds(base, CH)], src_v[b], si[b])
        pltpu.async_copy(dst_hbm.at[pl.ds(base, CH)], dst_v[b], si[b])

    def _idx_wait(b):
        pltpu.make_async_copy(src_hbm.at[pl.ds(0, CH)], src_v[b], si[b]).wait()
        pltpu.make_async_copy(dst_hbm.at[pl.ds(0, CH)], dst_v[b], si[b]).wait()

    def _gather_start(i, b):
        base = edge_base + i * CH
        pltpu.async_copy(a_hbm.at[pl.ds(base, CH)], a_v[b], sa[b])
        pltpu.async_copy(xs_hbm.at[src_v[b]], g1_v[b], sg1[b])
        pltpu.async_copy(xd_hbm.at[dst_v[b]], g2_v[b], sg2[b])

    def _gather_wait(b):
        pltpu.make_async_copy(a_hbm.at[pl.ds(0, CH)], a_v[b], sa[b]).wait()
        pltpu.make_async_copy(xs_hbm.at[src_v[b]], g1_v[b], sg1[b]).wait()
        pltpu.make_async_copy(xd_hbm.at[dst_v[b]], g2_v[b], sg2[b]).wait()

    def _wb_start(i, b):
        base = edge_base + i * CH
        pltpu.async_copy(a_v[b], ee_hbm.at[pl.ds(off + base, CH)], we[b])
        pltpu.async_copy(a_v[b], agg_sh.at[dst_v[b]], ws[b], add=True)

    def _wb_wait(b):
        pltpu.make_async_copy(a_v[b], ee_hbm.at[pl.ds(0, CH)], we[b]).wait()
        pltpu.make_async_copy(a_v[b], agg_sh.at[dst_v[b]], ws[b]).wait()

    def _compute(b, nrows):
        ab, g1b, g2b = a_v[b], g1_v[b], g2_v[b]

        def _row2(r2, carry2):
            r = r2 * 2
            for dr in range(2):
                for cc in range(D // 16):
                    sl = pl.ds(cc * 16, 16)
                    ab[r + dr, sl] = jnp.maximum(
                        ab[r + dr, sl] + g1b[r + dr, sl] + g2b[r + dr, sl],
                        0.0)
            return carry2

        lax.fori_loop(0, nrows // 2, _row2, 0)
        if nrows % 2:
            r = nrows - 1
            for cc in range(D // 16):
                sl = pl.ds(cc * 16, 16)
                ab[r, sl] = jnp.maximum(
                    ab[r, sl] + g1b[r, sl] + g2b[r, sl], 0.0)

    _idx_start(0, 0)
    _idx_wait(0)
    _gather_start(0, 0)

    @pl.loop(0, NCHUNK, step=2)
    def _pair(i0):
        for b in (0, 1):
            i = i0 + b
            nxt = 1 - b

            @pl.when(i > 0)
            def _():
                _wb_wait(nxt)

            @pl.when(i + 1 < NCHUNK)
            def _():
                _idx_start(i + 1, nxt)

            _gather_wait(b)

            @pl.when(i + 1 < NCHUNK)
            def _():
                _idx_wait(nxt)
                _gather_start(i + 1, nxt)

            _compute(b, CH)
            _wb_start(i, b)

    _wb_wait((NCHUNK - 1) % 2)

    if TAILE:
        base = edge_base + NCHUNK * CH
        pltpu.sync_copy(src_hbm.at[pl.ds(base, TAILE)], tsrc_v)
        pltpu.sync_copy(dst_hbm.at[pl.ds(base, TAILE)], tdst_v)
        pltpu.sync_copy(a_hbm.at[pl.ds(base, TAILE)],
                        a_v0.at[pl.ds(0, TAILE)])
        pltpu.async_copy(xs_hbm.at[tsrc_v], g1_v0.at[pl.ds(0, TAILE)],
                         sg10).wait()
        pltpu.async_copy(xd_hbm.at[tdst_v], g2_v0.at[pl.ds(0, TAILE)],
                         sg20).wait()
        _compute(0, TAILE)
        pltpu.sync_copy(a_v0.at[pl.ds(0, TAILE)],
                        ee_hbm.at[pl.ds(off + base, TAILE)])
        pltpu.sync_copy(a_v0.at[pl.ds(0, TAILE)], agg_sh.at[tdst_v],
                        add=True)

    plsc.subcore_barrier()
    pltpu.sync_copy(agg_sh.at[pl.ds(s * STRIPE, STRIPE)],
                    aggp_hbm.at[c, pl.ds(s * STRIPE, STRIPE)])

    @pl.when(s == NS - 1)
    def _dump_tail():
        pltpu.sync_copy(agg_sh.at[pl.ds(NS * STRIPE, TAIL)],
                        aggp_hbm.at[c, pl.ds(NS * STRIPE, TAIL)])


_SC_SCRATCH = (
    [pltpu.VMEM((CH,), jnp.int32),
     pltpu.VMEM((CH,), jnp.int32),
     pltpu.VMEM((CH, D), jnp.float32),
     pltpu.VMEM((CH, D), jnp.float32),
     pltpu.VMEM((CH, D), jnp.float32)] * 2
    + [pltpu.VMEM((TAILE, ), jnp.int32),
       pltpu.VMEM((TAILE, ), jnp.int32),
       pltpu.VMEM_SHARED((N, D), jnp.float32)]
    + [pltpu.SemaphoreType.DMA] * 12
)


def _sc_edge_first(a, xs, xd, src, dst):
    k = pl.kernel(
        functools.partial(_sc_edge_body, 0),
        mesh=plsc.VectorSubcoreMesh(core_axis_name="c", subcore_axis_name="s"),
        out_type=[
            jax.ShapeDtypeStruct((E, D), jnp.float32),
            jax.ShapeDtypeStruct((NC, N, D), jnp.float32),
        ],
        scratch_types=_SC_SCRATCH,
    )
    return k(a, xs, xd, src, dst)


def _sc_edge_second(a, xs, xd, src, dst, ee):
    def body(a_hbm, xs_hbm, xd_hbm, src_hbm, dst_hbm, ee_hbm, aggp_hbm,
             *rest):
        return _sc_edge_body(E2, a_hbm, xs_hbm, xd_hbm, src_hbm, dst_hbm,
                             ee_hbm, aggp_hbm, *rest)

    k = pl.kernel(
        body,
        mesh=plsc.VectorSubcoreMesh(core_axis_name="c", subcore_axis_name="s"),
        out_type=[
            jax.ShapeDtypeStruct((NC, N, D), jnp.float32),
        ],
        scratch_types=_SC_SCRATCH,
    )
    out = k(a, xs, xd, src, dst, ee)
    return out[0] if isinstance(out, (list, tuple)) else out


_BN = 1000


def _node_body(xn_ref, a0_ref, a1_ref, a2_ref, a3_ref, wn2_ref, gb_ref,
               b_ref, ne_ref, ge_ref):
    i = pl.program_id(0)
    agg = (a0_ref[...] + a1_ref[...]) + (a2_ref[...] + a3_ref[...])
    bvec = b_ref[0, 0, :]
    seg = jax.lax.broadcasted_iota(jnp.int32, (_BN, G), 1)
    mask = seg == bvec[:, None]
    gbb = jnp.dot(mask.astype(jnp.float32), gb_ref[...],
                  preferred_element_type=jnp.float32)
    ne = xn_ref[...] + jnp.dot(agg, wn2_ref[...],
                               preferred_element_type=jnp.float32) + gbb
    ne = jnp.maximum(ne, 0.0)
    ne_ref[...] = ne

    @pl.when(i == 0)
    def _():
        ge_ref[...] = jnp.full((G, D), -jnp.inf, jnp.float32)

    rows = [
        jnp.max(jnp.where(mask[:, g:g + 1], ne, -jnp.inf), axis=0,
                keepdims=True)
        for g in range(G)
    ]
    ge_ref[...] = jnp.maximum(ge_ref[...], jnp.concatenate(rows, axis=0))


def _node(xn, aggs, Wn2, gb, batch3):
    grid = N // _BN
    return pl.pallas_call(
        _node_body,
        grid=(grid,),
        in_specs=[
            pl.BlockSpec((_BN, D), lambda i: (i, 0)),
            pl.BlockSpec((_BN, D), lambda i: (i, 0)),
            pl.BlockSpec((_BN, D), lambda i: (i, 0)),
            pl.BlockSpec((_BN, D), lambda i: (i, 0)),
            pl.BlockSpec((_BN, D), lambda i: (i, 0)),
            pl.BlockSpec((D, D), lambda i: (0, 0)),
            pl.BlockSpec((G, D), lambda i: (0, 0)),
            pl.BlockSpec((1, 1, _BN), lambda i: (i, 0, 0)),
        ],
        out_specs=[
            pl.BlockSpec((_BN, D), lambda i: (i, 0)),
            pl.BlockSpec((G, D), lambda i: (0, 0)),
        ],
        out_shape=[
            jax.ShapeDtypeStruct((N, D), jnp.float32),
            jax.ShapeDtypeStruct((G, D), jnp.float32),
        ],
        compiler_params=pltpu.CompilerParams(
            dimension_semantics=("arbitrary",)),
    )(xn, aggs[0], aggs[1], aggs[2], aggs[3], Wn2, gb, batch3)


def kernel(x, edge_index, edge_attr, graph_attr, batch,
           W_msg, b_msg, W_node, b_node):
    src = edge_index[0]
    dst = edge_index[1]
    xs, xd, xn, gb = _prep(x, graph_attr, W_msg, W_node, b_msg, b_node)
    W3 = W_msg[2 * D:3 * D, :]
    a0 = _edge_mm(edge_attr[:E2], W3)
    ee, aggp0 = _sc_edge_first(a0, xs, xd, src[:E2], dst[:E2])
    a1 = _edge_mm(edge_attr[E2:], W3)
    aggp1 = _sc_edge_second(a1, xs, xd, src[E2:], dst[E2:], ee)
    ne, ge = _node(xn, (aggp0[0], aggp0[1], aggp1[0], aggp1[1]),
                   W_node[D:2 * D, :], gb,
                   batch.reshape(N // _BN, 1, _BN))
    return ne, ee, ge


# speedup vs baseline: 1.1825x; 1.1823x over previous
import functools

import jax
import jax.numpy as jnp
from jax import lax
from jax.experimental import pallas as pl
from jax.experimental.pallas import tpu as pltpu
from jax.experimental.pallas import tpu_sc as plsc

N = 10000
E = 320000
D = 128
G = 16

NC = 2
NS = 16
NW = NC * NS
EW = E // NW
CH = 64
NCHUNK = EW // CH
TAILE = EW - NCHUNK * CH
STRIPE = 624
TAIL = N - NS * STRIPE


def _prep_body(x_ref, ga_ref, wmsg_ref, wnode_ref, bmsg_ref, bnode_ref,
               xs_ref, xd_ref, xn_ref, gb_ref):
    x = x_ref[...]
    xs_ref[...] = jnp.dot(x, wmsg_ref[0:D, :], preferred_element_type=jnp.float32)
    xd_ref[...] = jnp.dot(x, wmsg_ref[D:2 * D, :],
                          preferred_element_type=jnp.float32) + bmsg_ref[...]
    xn_ref[...] = jnp.dot(x, wnode_ref[0:D, :],
                          preferred_element_type=jnp.float32) + bnode_ref[...]
    gb_ref[...] = jnp.dot(ga_ref[...], wnode_ref[2 * D:3 * D, :],
                          preferred_element_type=jnp.float32)


def _prep(x, graph_attr, W_msg, W_node, b_msg, b_node):
    return pl.pallas_call(
        _prep_body,
        out_shape=[
            jax.ShapeDtypeStruct((N, D), jnp.float32),
            jax.ShapeDtypeStruct((N, D), jnp.float32),
            jax.ShapeDtypeStruct((N, D), jnp.float32),
            jax.ShapeDtypeStruct((G, D), jnp.float32),
        ],
    )(x, graph_attr, W_msg, W_node, b_msg.reshape(1, D), b_node.reshape(1, D))


_BE = 4000


def _edge_mm_body(ea_ref, w_ref, a_ref):
    a_ref[...] = jnp.dot(ea_ref[...].astype(jnp.bfloat16),
                         w_ref[...].astype(jnp.bfloat16),
                         preferred_element_type=jnp.float32)


def _edge_mm(edge_attr, W3):
    grid = E // _BE
    return pl.pallas_call(
        _edge_mm_body,
        grid=(grid,),
        in_specs=[
            pl.BlockSpec((_BE, D), lambda i: (i, 0)),
            pl.BlockSpec((D, D), lambda i: (0, 0)),
        ],
        out_specs=pl.BlockSpec((_BE, D), lambda i: (i, 0)),
        out_shape=jax.ShapeDtypeStruct((E, D), jnp.float32),
        compiler_params=pltpu.CompilerParams(
            dimension_semantics=("arbitrary",)),
    )(edge_attr, W3)


def _sc_edge_body(a_hbm, xs_hbm, xd_hbm, src_hbm, dst_hbm,
                  ee_hbm, aggp_hbm,
                  src_v0, dst_v0, a_v0, g1_v0, g2_v0,
                  src_v1, dst_v1, a_v1, g1_v1, g2_v1,
                  tsrc_v, tdst_v, agg_sh,
                  sa0, sg10, sg20, sa1, sg11, sg21,
                  we0, ws0, we1, ws1, si0, si1):
    c = lax.axis_index("c")
    s = lax.axis_index("s")
    wid = c * NS + s
    edge_base = wid * EW

    src_v = (src_v0, src_v1)
    dst_v = (dst_v0, dst_v1)
    a_v = (a_v0, a_v1)
    g1_v = (g1_v0, g1_v1)
    g2_v = (g2_v0, g2_v1)
    sa = (sa0, sa1)
    sg1 = (sg10, sg11)
    sg2 = (sg20, sg21)
    we = (we0, we1)
    ws = (ws0, ws1)
    si = (si0, si1)

    zv = jnp.zeros((16,), jnp.float32)

    def _zero_row(r, carry):
        for cc in range(D // 16):
            g1_v0[r, pl.ds(cc * 16, 16)] = zv
        return carry

    lax.fori_loop(0, CH, _zero_row, 0)
    for j in range(STRIPE // CH):
        pltpu.sync_copy(g1_v0, agg_sh.at[pl.ds(s * STRIPE + j * CH, CH)])
    rem = STRIPE - (STRIPE // CH) * CH
    if rem:
        pltpu.sync_copy(g1_v0.at[pl.ds(0, rem)],
                        agg_sh.at[pl.ds(s * STRIPE + STRIPE - rem, rem)])

    @pl.when(s == NS - 1)
    def _zero_tail():
        pltpu.sync_copy(g1_v0.at[pl.ds(0, TAIL)],
                        agg_sh.at[pl.ds(NS * STRIPE, TAIL)])

    plsc.subcore_barrier()

    def _idx_start(i, b):
        base = edge_base + i * CH
        pltpu.async_copy(src_hbm.at[pl.ds(base, CH)], src_v[b], si[b])
        pltpu.async_copy(dst_hbm.at[pl.ds(base, CH)], dst_v[b], si[b])

    def _idx_wait(b):
        pltpu.make_async_copy(src_hbm.at[pl.ds(0, CH)], src_v[b], si[b]).wait()
        pltpu.make_async_copy(dst_hbm.at[pl.ds(0, CH)], dst_v[b], si[b]).wait()

    def _gather_start(i, b):
        base = edge_base + i * CH
        pltpu.async_copy(a_hbm.at[pl.ds(base, CH)], a_v[b], sa[b])
        pltpu.async_copy(xs_hbm.at[src_v[b]], g1_v[b], sg1[b])
        pltpu.async_copy(xd_hbm.at[dst_v[b]], g2_v[b], sg2[b])

    def _gather_wait(b):
        pltpu.make_async_copy(a_hbm.at[pl.ds(0, CH)], a_v[b], sa[b]).wait()
        pltpu.make_async_copy(xs_hbm.at[src_v[b]], g1_v[b], sg1[b]).wait()
        pltpu.make_async_copy(xd_hbm.at[dst_v[b]], g2_v[b], sg2[b]).wait()

    def _wb_start(i, b):
        base = edge_base + i * CH
        pltpu.async_copy(a_v[b], ee_hbm.at[pl.ds(base, CH)], we[b])
        pltpu.async_copy(a_v[b], agg_sh.at[dst_v[b]], ws[b], add=True)

    def _wb_wait(b):
        pltpu.make_async_copy(a_v[b], ee_hbm.at[pl.ds(0, CH)], we[b]).wait()
        pltpu.make_async_copy(a_v[b], agg_sh.at[dst_v[b]], ws[b]).wait()

    def _compute(b, nrows):
        ab, g1b, g2b = a_v[b], g1_v[b], g2_v[b]

        def _row2(r2, carry2):
            r = r2 * 2
            for dr in range(2):
                for cc in range(D // 16):
                    sl = pl.ds(cc * 16, 16)
                    ab[r + dr, sl] = jnp.maximum(
                        ab[r + dr, sl] + g1b[r + dr, sl] + g2b[r + dr, sl],
                        0.0)
            return carry2

        lax.fori_loop(0, nrows // 2, _row2, 0)
        if nrows % 2:
            r = nrows - 1
            for cc in range(D // 16):
                sl = pl.ds(cc * 16, 16)
                ab[r, sl] = jnp.maximum(
                    ab[r, sl] + g1b[r, sl] + g2b[r, sl], 0.0)

    _idx_start(0, 0)
    _idx_wait(0)
    _gather_start(0, 0)

    @pl.loop(0, NCHUNK, step=2)
    def _pair(i0):
        for b in (0, 1):
            i = i0 + b
            nxt = 1 - b

            @pl.when(i > 0)
            def _():
                _wb_wait(nxt)

            @pl.when(i + 1 < NCHUNK)
            def _():
                _idx_start(i + 1, nxt)

            _gather_wait(b)

            @pl.when(i + 1 < NCHUNK)
            def _():
                _idx_wait(nxt)
                _gather_start(i + 1, nxt)

            _compute(b, CH)
            _wb_start(i, b)

    _wb_wait((NCHUNK - 1) % 2)

    if TAILE:
        base = edge_base + NCHUNK * CH
        pltpu.sync_copy(src_hbm.at[pl.ds(base, TAILE)], tsrc_v)
        pltpu.sync_copy(dst_hbm.at[pl.ds(base, TAILE)], tdst_v)
        pltpu.sync_copy(a_hbm.at[pl.ds(base, TAILE)],
                        a_v0.at[pl.ds(0, TAILE)])
        pltpu.async_copy(xs_hbm.at[tsrc_v], g1_v0.at[pl.ds(0, TAILE)],
                         sg10).wait()
        pltpu.async_copy(xd_hbm.at[tdst_v], g2_v0.at[pl.ds(0, TAILE)],
                         sg20).wait()
        _compute(0, TAILE)
        pltpu.sync_copy(a_v0.at[pl.ds(0, TAILE)],
                        ee_hbm.at[pl.ds(base, TAILE)])
        pltpu.sync_copy(a_v0.at[pl.ds(0, TAILE)], agg_sh.at[tdst_v],
                        add=True)

    plsc.subcore_barrier()
    pltpu.sync_copy(agg_sh.at[pl.ds(s * STRIPE, STRIPE)],
                    aggp_hbm.at[c, pl.ds(s * STRIPE, STRIPE)])

    @pl.when(s == NS - 1)
    def _dump_tail():
        pltpu.sync_copy(agg_sh.at[pl.ds(NS * STRIPE, TAIL)],
                        aggp_hbm.at[c, pl.ds(NS * STRIPE, TAIL)])


def _sc_edge(a, xs, xd, src, dst):
    k = pl.kernel(
        _sc_edge_body,
        mesh=plsc.VectorSubcoreMesh(core_axis_name="c", subcore_axis_name="s"),
        out_type=[
            jax.ShapeDtypeStruct((E, D), jnp.float32),
            jax.ShapeDtypeStruct((NC, N, D), jnp.float32),
        ],
        scratch_types=(
            [pltpu.VMEM((CH,), jnp.int32),
             pltpu.VMEM((CH,), jnp.int32),
             pltpu.VMEM((CH, D), jnp.float32),
             pltpu.VMEM((CH, D), jnp.float32),
             pltpu.VMEM((CH, D), jnp.float32)] * 2
            + [pltpu.VMEM((TAILE, ), jnp.int32),
               pltpu.VMEM((TAILE, ), jnp.int32),
               pltpu.VMEM_SHARED((N, D), jnp.float32)]
            + [pltpu.SemaphoreType.DMA] * 12
        ),
    )
    return k(a, xs, xd, src, dst)


_BN = 1000


def _node_body(xn_ref, a0_ref, a1_ref, wn2_ref, gb_ref, b_ref,
               ne_ref, ge_ref):
    i = pl.program_id(0)
    agg = a0_ref[...] + a1_ref[...]
    bvec = b_ref[0, 0, :]
    seg = jax.lax.broadcasted_iota(jnp.int32, (_BN, G), 1)
    mask = seg == bvec[:, None]
    gbb = jnp.dot(mask.astype(jnp.float32), gb_ref[...],
                  preferred_element_type=jnp.float32)
    ne = xn_ref[...] + jnp.dot(agg, wn2_ref[...],
                               preferred_element_type=jnp.float32) + gbb
    ne = jnp.maximum(ne, 0.0)
    ne_ref[...] = ne

    @pl.when(i == 0)
    def _():
        ge_ref[...] = jnp.full((G, D), -jnp.inf, jnp.float32)

    rows = [
        jnp.max(jnp.where(mask[:, g:g + 1], ne, -jnp.inf), axis=0,
                keepdims=True)
        for g in range(G)
    ]
    ge_ref[...] = jnp.maximum(ge_ref[...], jnp.concatenate(rows, axis=0))


def _node(xn, agg0, agg1, Wn2, gb, batch3):
    grid = N // _BN
    return pl.pallas_call(
        _node_body,
        grid=(grid,),
        in_specs=[
            pl.BlockSpec((_BN, D), lambda i: (i, 0)),
            pl.BlockSpec((_BN, D), lambda i: (i, 0)),
            pl.BlockSpec((_BN, D), lambda i: (i, 0)),
            pl.BlockSpec((D, D), lambda i: (0, 0)),
            pl.BlockSpec((G, D), lambda i: (0, 0)),
            pl.BlockSpec((1, 1, _BN), lambda i: (i, 0, 0)),
        ],
        out_specs=[
            pl.BlockSpec((_BN, D), lambda i: (i, 0)),
            pl.BlockSpec((G, D), lambda i: (0, 0)),
        ],
        out_shape=[
            jax.ShapeDtypeStruct((N, D), jnp.float32),
            jax.ShapeDtypeStruct((G, D), jnp.float32),
        ],
        compiler_params=pltpu.CompilerParams(
            dimension_semantics=("arbitrary",)),
    )(xn, agg0, agg1, Wn2, gb, batch3)


def kernel(x, edge_index, edge_attr, graph_attr, batch,
           W_msg, b_msg, W_node, b_node):
    src = edge_index[0]
    dst = edge_index[1]
    xs, xd, xn, gb = _prep(x, graph_attr, W_msg, W_node, b_msg, b_node)
    a = _edge_mm(edge_attr, W_msg[2 * D:3 * D, :])
    ee, aggp = _sc_edge(a, xs, xd, src, dst)
    ne, ge = _node(xn, aggp[0], aggp[1], W_node[D:2 * D, :], gb,
                   batch.reshape(N // _BN, 1, _BN))
    return ne, ee, ge


# speedup vs baseline: 1.1833x; 1.0007x over previous
import functools

import jax
import jax.numpy as jnp
from jax import lax
from jax.experimental import pallas as pl
from jax.experimental.pallas import tpu as pltpu
from jax.experimental.pallas import tpu_sc as plsc

N = 10000
E = 320000
D = 128
G = 16

NC = 2
NS = 16
NW = NC * NS
EW = E // NW
CH = 64
NCHUNK = EW // CH
TAILE = EW - NCHUNK * CH
STRIPE = 624
TAIL = N - NS * STRIPE


def _prep_body(x_ref, ga_ref, wmsg_ref, wnode_ref, bmsg_ref, bnode_ref,
               xs_ref, xd_ref, xn_ref, gb_ref):
    x = x_ref[...]
    xs_ref[...] = jnp.dot(x, wmsg_ref[0:D, :], preferred_element_type=jnp.float32)
    xd_ref[...] = jnp.dot(x, wmsg_ref[D:2 * D, :],
                          preferred_element_type=jnp.float32) + bmsg_ref[...]
    xn_ref[...] = jnp.dot(x, wnode_ref[0:D, :],
                          preferred_element_type=jnp.float32) + bnode_ref[...]
    gb_ref[...] = jnp.dot(ga_ref[...], wnode_ref[2 * D:3 * D, :],
                          preferred_element_type=jnp.float32)


def _prep(x, graph_attr, W_msg, W_node, b_msg, b_node):
    return pl.pallas_call(
        _prep_body,
        out_shape=[
            jax.ShapeDtypeStruct((N, D), jnp.float32),
            jax.ShapeDtypeStruct((N, D), jnp.float32),
            jax.ShapeDtypeStruct((N, D), jnp.float32),
            jax.ShapeDtypeStruct((G, D), jnp.float32),
        ],
    )(x, graph_attr, W_msg, W_node, b_msg.reshape(1, D), b_node.reshape(1, D))


_BE = 4000


def _edge_mm_body(ea_ref, w_ref, a_ref):
    a_ref[...] = jnp.dot(ea_ref[...], w_ref[...],
                         preferred_element_type=jnp.float32)


def _edge_mm(edge_attr, W3):
    grid = E // _BE
    return pl.pallas_call(
        _edge_mm_body,
        grid=(grid,),
        in_specs=[
            pl.BlockSpec((_BE, D), lambda i: (i, 0)),
            pl.BlockSpec((D, D), lambda i: (0, 0)),
        ],
        out_specs=pl.BlockSpec((_BE, D), lambda i: (i, 0)),
        out_shape=jax.ShapeDtypeStruct((E, D), jnp.float32),
        compiler_params=pltpu.CompilerParams(
            dimension_semantics=("arbitrary",)),
    )(edge_attr, W3)


def _sc_edge_body(a_hbm, xs_hbm, xd_hbm, src_hbm, dst_hbm,
                  ee_hbm, aggp_hbm,
                  src_v0, dst_v0, a_v0, g1_v0, g2_v0,
                  src_v1, dst_v1, a_v1, g1_v1, g2_v1,
                  tsrc_v, tdst_v, agg_sh,
                  sa0, sg10, sg20, sa1, sg11, sg21,
                  we0, ws0, we1, ws1, si0, si1):
    c = lax.axis_index("c")
    s = lax.axis_index("s")
    wid = c * NS + s
    edge_base = wid * EW

    src_v = (src_v0, src_v1)
    dst_v = (dst_v0, dst_v1)
    a_v = (a_v0, a_v1)
    g1_v = (g1_v0, g1_v1)
    g2_v = (g2_v0, g2_v1)
    sa = (sa0, sa1)
    sg1 = (sg10, sg11)
    sg2 = (sg20, sg21)
    we = (we0, we1)
    ws = (ws0, ws1)
    si = (si0, si1)

    zv = jnp.zeros((16,), jnp.float32)

    def _zero_row(r, carry):
        for cc in range(D // 16):
            g1_v0[r, pl.ds(cc * 16, 16)] = zv
        return carry

    lax.fori_loop(0, CH, _zero_row, 0)
    for j in range(STRIPE // CH):
        pltpu.sync_copy(g1_v0, agg_sh.at[pl.ds(s * STRIPE + j * CH, CH)])
    rem = STRIPE - (STRIPE // CH) * CH
    if rem:
        pltpu.sync_copy(g1_v0.at[pl.ds(0, rem)],
                        agg_sh.at[pl.ds(s * STRIPE + STRIPE - rem, rem)])

    @pl.when(s == NS - 1)
    def _zero_tail():
        pltpu.sync_copy(g1_v0.at[pl.ds(0, TAIL)],
                        agg_sh.at[pl.ds(NS * STRIPE, TAIL)])

    plsc.subcore_barrier()

    def _idx_start(i, b):
        base = edge_base + i * CH
        pltpu.async_copy(src_hbm.at[pl.ds(base, CH)], src_v[b], si[b])
        pltpu.async_copy(dst_hbm.at[pl.ds(base, CH)], dst_v[b], si[b])

    def _idx_wait(b):
        pltpu.make_async_copy(src_hbm.at[pl.ds(0, CH)], src_v[b], si[b]).wait()
        pltpu.make_async_copy(dst_hbm.at[pl.ds(0, CH)], dst_v[b], si[b]).wait()

    def _gather_start(i, b):
        base = edge_base + i * CH
        pltpu.async_copy(a_hbm.at[pl.ds(base, CH)], a_v[b], sa[b])
        pltpu.async_copy(xs_hbm.at[src_v[b]], g1_v[b], sg1[b])
        pltpu.async_copy(xd_hbm.at[dst_v[b]], g2_v[b], sg2[b])

    def _gather_wait(b):
        pltpu.make_async_copy(a_hbm.at[pl.ds(0, CH)], a_v[b], sa[b]).wait()
        pltpu.make_async_copy(xs_hbm.at[src_v[b]], g1_v[b], sg1[b]).wait()
        pltpu.make_async_copy(xd_hbm.at[dst_v[b]], g2_v[b], sg2[b]).wait()

    def _wb_start(i, b):
        base = edge_base + i * CH
        pltpu.async_copy(a_v[b], ee_hbm.at[pl.ds(base, CH)], we[b])
        pltpu.async_copy(a_v[b], agg_sh.at[dst_v[b]], ws[b], add=True)

    def _wb_wait(b):
        pltpu.make_async_copy(a_v[b], ee_hbm.at[pl.ds(0, CH)], we[b]).wait()
        pltpu.make_async_copy(a_v[b], agg_sh.at[dst_v[b]], ws[b]).wait()

    def _compute(b, nrows):
        ab, g1b, g2b = a_v[b], g1_v[b], g2_v[b]

        def _row2(r2, carry2):
            r = r2 * 2
            for dr in range(2):
                for cc in range(D // 16):
                    sl = pl.ds(cc * 16, 16)
                    ab[r + dr, sl] = jnp.maximum(
                        ab[r + dr, sl] + g1b[r + dr, sl] + g2b[r + dr, sl],
                        0.0)
            return carry2

        lax.fori_loop(0, nrows // 2, _row2, 0)
        if nrows % 2:
            r = nrows - 1
            for cc in range(D // 16):
                sl = pl.ds(cc * 16, 16)
                ab[r, sl] = jnp.maximum(
                    ab[r, sl] + g1b[r, sl] + g2b[r, sl], 0.0)

    _idx_start(0, 0)
    _idx_wait(0)
    _gather_start(0, 0)

    @pl.loop(0, NCHUNK, step=2)
    def _pair(i0):
        for b in (0, 1):
            i = i0 + b
            nxt = 1 - b

            @pl.when(i > 0)
            def _():
                _wb_wait(nxt)

            @pl.when(i + 1 < NCHUNK)
            def _():
                _idx_start(i + 1, nxt)

            _gather_wait(b)

            @pl.when(i + 1 < NCHUNK)
            def _():
                _idx_wait(nxt)
                _gather_start(i + 1, nxt)

            _compute(b, CH)
            _wb_start(i, b)

    _wb_wait((NCHUNK - 1) % 2)

    if TAILE:
        base = edge_base + NCHUNK * CH
        pltpu.sync_copy(src_hbm.at[pl.ds(base, TAILE)], tsrc_v)
        pltpu.sync_copy(dst_hbm.at[pl.ds(base, TAILE)], tdst_v)
        pltpu.sync_copy(a_hbm.at[pl.ds(base, TAILE)],
                        a_v0.at[pl.ds(0, TAILE)])
        pltpu.async_copy(xs_hbm.at[tsrc_v], g1_v0.at[pl.ds(0, TAILE)],
                         sg10).wait()
        pltpu.async_copy(xd_hbm.at[tdst_v], g2_v0.at[pl.ds(0, TAILE)],
                         sg20).wait()
        _compute(0, TAILE)
        pltpu.sync_copy(a_v0.at[pl.ds(0, TAILE)],
                        ee_hbm.at[pl.ds(base, TAILE)])
        pltpu.sync_copy(a_v0.at[pl.ds(0, TAILE)], agg_sh.at[tdst_v],
                        add=True)

    plsc.subcore_barrier()
    pltpu.sync_copy(agg_sh.at[pl.ds(s * STRIPE, STRIPE)],
                    aggp_hbm.at[c, pl.ds(s * STRIPE, STRIPE)])

    @pl.when(s == NS - 1)
    def _dump_tail():
        pltpu.sync_copy(agg_sh.at[pl.ds(NS * STRIPE, TAIL)],
                        aggp_hbm.at[c, pl.ds(NS * STRIPE, TAIL)])


def _sc_edge(a, xs, xd, src, dst):
    k = pl.kernel(
        _sc_edge_body,
        mesh=plsc.VectorSubcoreMesh(core_axis_name="c", subcore_axis_name="s"),
        out_type=[
            jax.ShapeDtypeStruct((E, D), jnp.float32),
            jax.ShapeDtypeStruct((NC, N, D), jnp.float32),
        ],
        scratch_types=(
            [pltpu.VMEM((CH,), jnp.int32),
             pltpu.VMEM((CH,), jnp.int32),
             pltpu.VMEM((CH, D), jnp.float32),
             pltpu.VMEM((CH, D), jnp.float32),
             pltpu.VMEM((CH, D), jnp.float32)] * 2
            + [pltpu.VMEM((TAILE, ), jnp.int32),
               pltpu.VMEM((TAILE, ), jnp.int32),
               pltpu.VMEM_SHARED((N, D), jnp.float32)]
            + [pltpu.SemaphoreType.DMA] * 12
        ),
    )
    return k(a, xs, xd, src, dst)


_BN = 1000


def _node_body(xn_ref, a0_ref, a1_ref, wn2_ref, gb_ref, b_ref,
               ne_ref, ge_ref):
    i = pl.program_id(0)
    agg = a0_ref[...] + a1_ref[...]
    bvec = b_ref[0, 0, :]
    seg = jax.lax.broadcasted_iota(jnp.int32, (_BN, G), 1)
    mask = seg == bvec[:, None]
    gbb = jnp.dot(mask.astype(jnp.float32), gb_ref[...],
                  preferred_element_type=jnp.float32)
    ne = xn_ref[...] + jnp.dot(agg, wn2_ref[...],
                               preferred_element_type=jnp.float32) + gbb
    ne = jnp.maximum(ne, 0.0)
    ne_ref[...] = ne

    @pl.when(i == 0)
    def _():
        ge_ref[...] = jnp.full((G, D), -jnp.inf, jnp.float32)

    rows = [
        jnp.max(jnp.where(mask[:, g:g + 1], ne, -jnp.inf), axis=0,
                keepdims=True)
        for g in range(G)
    ]
    ge_ref[...] = jnp.maximum(ge_ref[...], jnp.concatenate(rows, axis=0))


def _node(xn, agg0, agg1, Wn2, gb, batch3):
    grid = N // _BN
    return pl.pallas_call(
        _node_body,
        grid=(grid,),
        in_specs=[
            pl.BlockSpec((_BN, D), lambda i: (i, 0)),
            pl.BlockSpec((_BN, D), lambda i: (i, 0)),
            pl.BlockSpec((_BN, D), lambda i: (i, 0)),
            pl.BlockSpec((D, D), lambda i: (0, 0)),
            pl.BlockSpec((G, D), lambda i: (0, 0)),
            pl.BlockSpec((1, 1, _BN), lambda i: (i, 0, 0)),
        ],
        out_specs=[
            pl.BlockSpec((_BN, D), lambda i: (i, 0)),
            pl.BlockSpec((G, D), lambda i: (0, 0)),
        ],
        out_shape=[
            jax.ShapeDtypeStruct((N, D), jnp.float32),
            jax.ShapeDtypeStruct((G, D), jnp.float32),
        ],
        compiler_params=pltpu.CompilerParams(
            dimension_semantics=("arbitrary",)),
    )(xn, agg0, agg1, Wn2, gb, batch3)


def kernel(x, edge_index, edge_attr, graph_attr, batch,
           W_msg, b_msg, W_node, b_node):
    src = edge_index[0]
    dst = edge_index[1]
    xs, xd, xn, gb = _prep(x, graph_attr, W_msg, W_node, b_msg, b_node)
    a = _edge_mm(edge_attr, W_msg[2 * D:3 * D, :])
    ee, aggp = _sc_edge(a, xs, xd, src, dst)
    ne, ge = _node(xn, aggp[0], aggp[1], W_node[D:2 * D, :], gb,
                   batch.reshape(N // _BN, 1, _BN))
    return ne, ee, ge


# speedup vs baseline: 1.2134x; 1.0254x over previous
import functools

import jax
import jax.numpy as jnp
from jax import lax
from jax.experimental import pallas as pl
from jax.experimental.pallas import tpu as pltpu
from jax.experimental.pallas import tpu_sc as plsc

N = 10000
E = 320000
D = 128
G = 16

NC = 2
NS = 16
NW = NC * NS
EW = E // NW
CH = 64
NCHUNK = EW // CH
TAILE = EW - NCHUNK * CH
STRIPE = 624
TAIL = N - NS * STRIPE


def _prep_body(x_ref, ga_ref, wmsg_ref, wnode_ref, bmsg_ref, bnode_ref,
               xs_ref, xd_ref, xn_ref, gb_ref):
    x = x_ref[...]
    xs_ref[...] = jnp.dot(x, wmsg_ref[0:D, :], preferred_element_type=jnp.float32)
    xd_ref[...] = jnp.dot(x, wmsg_ref[D:2 * D, :],
                          preferred_element_type=jnp.float32) + bmsg_ref[...]
    xn_ref[...] = jnp.dot(x, wnode_ref[0:D, :],
                          preferred_element_type=jnp.float32) + bnode_ref[...]
    gb_ref[...] = jnp.dot(ga_ref[...], wnode_ref[2 * D:3 * D, :],
                          preferred_element_type=jnp.float32)


def _prep(x, graph_attr, W_msg, W_node, b_msg, b_node):
    return pl.pallas_call(
        _prep_body,
        out_shape=[
            jax.ShapeDtypeStruct((N, D), jnp.float32),
            jax.ShapeDtypeStruct((N, D), jnp.float32),
            jax.ShapeDtypeStruct((N, D), jnp.float32),
            jax.ShapeDtypeStruct((G, D), jnp.float32),
        ],
    )(x, graph_attr, W_msg, W_node, b_msg.reshape(1, D), b_node.reshape(1, D))


_BE = 8000


def _edge_mm_body(ea_ref, w_ref, a_ref):
    a_ref[...] = jnp.dot(ea_ref[...], w_ref[...],
                         preferred_element_type=jnp.float32)


def _edge_mm(edge_attr, W3):
    grid = E // _BE
    return pl.pallas_call(
        _edge_mm_body,
        grid=(grid,),
        in_specs=[
            pl.BlockSpec((_BE, D), lambda i: (i, 0)),
            pl.BlockSpec((D, D), lambda i: (0, 0)),
        ],
        out_specs=pl.BlockSpec((_BE, D), lambda i: (i, 0)),
        out_shape=jax.ShapeDtypeStruct((E, D), jnp.float32),
        compiler_params=pltpu.CompilerParams(
            dimension_semantics=("arbitrary",)),
    )(edge_attr, W3)


def _sc_edge_body(a_hbm, xs_hbm, xd_hbm, src_hbm, dst_hbm,
                  ee_hbm, aggp_hbm,
                  src_v0, dst_v0, a_v0, g1_v0, g2_v0,
                  src_v1, dst_v1, a_v1, g1_v1, g2_v1,
                  tsrc_v, tdst_v, agg_sh,
                  sa0, sg10, sg20, sa1, sg11, sg21,
                  we0, ws0, we1, ws1, si0, si1):
    c = lax.axis_index("c")
    s = lax.axis_index("s")
    wid = c * NS + s
    edge_base = wid * EW

    src_v = (src_v0, src_v1)
    dst_v = (dst_v0, dst_v1)
    a_v = (a_v0, a_v1)
    g1_v = (g1_v0, g1_v1)
    g2_v = (g2_v0, g2_v1)
    sa = (sa0, sa1)
    sg1 = (sg10, sg11)
    sg2 = (sg20, sg21)
    we = (we0, we1)
    ws = (ws0, ws1)
    si = (si0, si1)

    zv = jnp.zeros((16,), jnp.float32)

    def _zero_row(r, carry):
        for cc in range(D // 16):
            g1_v0[r, pl.ds(cc * 16, 16)] = zv
        return carry

    lax.fori_loop(0, CH, _zero_row, 0)
    for j in range(STRIPE // CH):
        pltpu.sync_copy(g1_v0, agg_sh.at[pl.ds(s * STRIPE + j * CH, CH)])
    rem = STRIPE - (STRIPE // CH) * CH
    if rem:
        pltpu.sync_copy(g1_v0.at[pl.ds(0, rem)],
                        agg_sh.at[pl.ds(s * STRIPE + STRIPE - rem, rem)])

    @pl.when(s == NS - 1)
    def _zero_tail():
        pltpu.sync_copy(g1_v0.at[pl.ds(0, TAIL)],
                        agg_sh.at[pl.ds(NS * STRIPE, TAIL)])

    plsc.subcore_barrier()

    def _idx_start(i, b):
        base = edge_base + i * CH
        pltpu.async_copy(src_hbm.at[pl.ds(base, CH)], src_v[b], si[b])
        pltpu.async_copy(dst_hbm.at[pl.ds(base, CH)], dst_v[b], si[b])

    def _idx_wait(b):
        pltpu.make_async_copy(src_hbm.at[pl.ds(0, CH)], src_v[b], si[b]).wait()
        pltpu.make_async_copy(dst_hbm.at[pl.ds(0, CH)], dst_v[b], si[b]).wait()

    def _gather_start(i, b):
        base = edge_base + i * CH
        pltpu.async_copy(a_hbm.at[pl.ds(base, CH)], a_v[b], sa[b])
        pltpu.async_copy(xs_hbm.at[src_v[b]], g1_v[b], sg1[b])
        pltpu.async_copy(xd_hbm.at[dst_v[b]], g2_v[b], sg2[b])

    def _gather_wait(b):
        pltpu.make_async_copy(a_hbm.at[pl.ds(0, CH)], a_v[b], sa[b]).wait()
        pltpu.make_async_copy(xs_hbm.at[src_v[b]], g1_v[b], sg1[b]).wait()
        pltpu.make_async_copy(xd_hbm.at[dst_v[b]], g2_v[b], sg2[b]).wait()

    def _wb_start(i, b):
        base = edge_base + i * CH
        pltpu.async_copy(a_v[b], ee_hbm.at[pl.ds(base, CH)], we[b])
        pltpu.async_copy(a_v[b], agg_sh.at[dst_v[b]], ws[b], add=True)

    def _wb_wait(b):
        pltpu.make_async_copy(a_v[b], ee_hbm.at[pl.ds(0, CH)], we[b]).wait()
        pltpu.make_async_copy(a_v[b], agg_sh.at[dst_v[b]], ws[b]).wait()

    def _compute(b, nrows):
        ab, g1b, g2b = a_v[b], g1_v[b], g2_v[b]

        def _row4(r4, carry2):
            r = r4 * 4
            for dr in range(4):
                for cc in range(D // 16):
                    sl = pl.ds(cc * 16, 16)
                    ab[r + dr, sl] = jnp.maximum(
                        ab[r + dr, sl] + g1b[r + dr, sl] + g2b[r + dr, sl],
                        0.0)
            return carry2

        lax.fori_loop(0, nrows // 4, _row4, 0)
        for r in range(nrows - nrows % 4, nrows):
            for cc in range(D // 16):
                sl = pl.ds(cc * 16, 16)
                ab[r, sl] = jnp.maximum(
                    ab[r, sl] + g1b[r, sl] + g2b[r, sl], 0.0)

    _idx_start(0, 0)
    _idx_wait(0)
    _gather_start(0, 0)

    @pl.loop(0, NCHUNK, step=2)
    def _pair(i0):
        for b in (0, 1):
            i = i0 + b
            nxt = 1 - b

            @pl.when(i > 0)
            def _():
                _wb_wait(nxt)

            @pl.when(i + 1 < NCHUNK)
            def _():
                _idx_start(i + 1, nxt)

            _gather_wait(b)

            @pl.when(i + 1 < NCHUNK)
            def _():
                _idx_wait(nxt)
                _gather_start(i + 1, nxt)

            _compute(b, CH)
            _wb_start(i, b)

    _wb_wait((NCHUNK - 1) % 2)

    if TAILE:
        base = edge_base + NCHUNK * CH
        pltpu.sync_copy(src_hbm.at[pl.ds(base, TAILE)], tsrc_v)
        pltpu.sync_copy(dst_hbm.at[pl.ds(base, TAILE)], tdst_v)
        pltpu.sync_copy(a_hbm.at[pl.ds(base, TAILE)],
                        a_v0.at[pl.ds(0, TAILE)])
        pltpu.async_copy(xs_hbm.at[tsrc_v], g1_v0.at[pl.ds(0, TAILE)],
                         sg10).wait()
        pltpu.async_copy(xd_hbm.at[tdst_v], g2_v0.at[pl.ds(0, TAILE)],
                         sg20).wait()
        _compute(0, TAILE)
        pltpu.sync_copy(a_v0.at[pl.ds(0, TAILE)],
                        ee_hbm.at[pl.ds(base, TAILE)])
        pltpu.sync_copy(a_v0.at[pl.ds(0, TAILE)], agg_sh.at[tdst_v],
                        add=True)

    plsc.subcore_barrier()
    pltpu.sync_copy(agg_sh.at[pl.ds(s * STRIPE, STRIPE)],
                    aggp_hbm.at[c, pl.ds(s * STRIPE, STRIPE)])

    @pl.when(s == NS - 1)
    def _dump_tail():
        pltpu.sync_copy(agg_sh.at[pl.ds(NS * STRIPE, TAIL)],
                        aggp_hbm.at[c, pl.ds(NS * STRIPE, TAIL)])


def _sc_edge(a, xs, xd, src, dst):
    k = pl.kernel(
        _sc_edge_body,
        mesh=plsc.VectorSubcoreMesh(core_axis_name="c", subcore_axis_name="s"),
        out_type=[
            jax.ShapeDtypeStruct((E, D), jnp.float32),
            jax.ShapeDtypeStruct((NC, N, D), jnp.float32),
        ],
        scratch_types=(
            [pltpu.VMEM((CH,), jnp.int32),
             pltpu.VMEM((CH,), jnp.int32),
             pltpu.VMEM((CH, D), jnp.float32),
             pltpu.VMEM((CH, D), jnp.float32),
             pltpu.VMEM((CH, D), jnp.float32)] * 2
            + [pltpu.VMEM((TAILE, ), jnp.int32),
               pltpu.VMEM((TAILE, ), jnp.int32),
               pltpu.VMEM_SHARED((N, D), jnp.float32)]
            + [pltpu.SemaphoreType.DMA] * 12
        ),
    )
    return k(a, xs, xd, src, dst)


_BN = 1000


def _node_body(xn_ref, a0_ref, a1_ref, wn2_ref, gb_ref, b_ref,
               ne_ref, ge_ref):
    i = pl.program_id(0)
    agg = a0_ref[...] + a1_ref[...]
    bvec = b_ref[0, 0, :]
    seg = jax.lax.broadcasted_iota(jnp.int32, (_BN, G), 1)
    mask = seg == bvec[:, None]
    gbb = jnp.dot(mask.astype(jnp.float32), gb_ref[...],
                  preferred_element_type=jnp.float32)
    ne = xn_ref[...] + jnp.dot(agg, wn2_ref[...],
                               preferred_element_type=jnp.float32) + gbb
    ne = jnp.maximum(ne, 0.0)
    ne_ref[...] = ne

    @pl.when(i == 0)
    def _():
        ge_ref[...] = jnp.full((G, D), -jnp.inf, jnp.float32)

    rows = [
        jnp.max(jnp.where(mask[:, g:g + 1], ne, -jnp.inf), axis=0,
                keepdims=True)
        for g in range(G)
    ]
    ge_ref[...] = jnp.maximum(ge_ref[...], jnp.concatenate(rows, axis=0))


def _node(xn, agg0, agg1, Wn2, gb, batch3):
    grid = N // _BN
    return pl.pallas_call(
        _node_body,
        grid=(grid,),
        in_specs=[
            pl.BlockSpec((_BN, D), lambda i: (i, 0)),
            pl.BlockSpec((_BN, D), lambda i: (i, 0)),
            pl.BlockSpec((_BN, D), lambda i: (i, 0)),
            pl.BlockSpec((D, D), lambda i: (0, 0)),
            pl.BlockSpec((G, D), lambda i: (0, 0)),
            pl.BlockSpec((1, 1, _BN), lambda i: (i, 0, 0)),
        ],
        out_specs=[
            pl.BlockSpec((_BN, D), lambda i: (i, 0)),
            pl.BlockSpec((G, D), lambda i: (0, 0)),
        ],
        out_shape=[
            jax.ShapeDtypeStruct((N, D), jnp.float32),
            jax.ShapeDtypeStruct((G, D), jnp.float32),
        ],
        compiler_params=pltpu.CompilerParams(
            dimension_semantics=("arbitrary",)),
    )(xn, agg0, agg1, Wn2, gb, batch3)


def kernel(x, edge_index, edge_attr, graph_attr, batch,
           W_msg, b_msg, W_node, b_node):
    src = edge_index[0]
    dst = edge_index[1]
    xs, xd, xn, gb = _prep(x, graph_attr, W_msg, W_node, b_msg, b_node)
    a = _edge_mm(edge_attr, W_msg[2 * D:3 * D, :])
    ee, aggp = _sc_edge(a, xs, xd, src, dst)
    ne, ge = _node(xn, aggp[0], aggp[1], W_node[D:2 * D, :], gb,
                   batch.reshape(N // _BN, 1, _BN))
    return ne, ee, ge
